# baseline (speedup 1.0000x reference)
import functools

import jax
import jax.numpy as jnp
from jax import lax
from jax.experimental import pallas as pl
from jax.experimental.pallas import tpu as pltpu
from jax.experimental.pallas import tpu_sc as plsc

_LANES = 16
_CHUNK = 32
_NBUF = 3
_RB = 8

_DNUMS = lax.GatherDimensionNumbers(
    offset_dims=(), collapsed_slice_dims=(0,), start_index_map=(0,))


def _xlane(v, idx):
    return lax.gather(v, idx[:, None], dimension_numbers=_DNUMS,
                      slice_sizes=(1,),
                      mode=lax.GatherScatterMode.PROMISE_IN_BOUNDS)


@functools.lru_cache(maxsize=None)
def _make_sc_kernel(n_nodes, d_feat, n_edges):
    info = plsc.get_sparse_core_info()
    nc, ns = info.num_cores, info.num_subcores
    nw = nc * ns
    assert n_edges % (nw * 8) == 0
    epw = n_edges // nw
    epw_c = -(-epw // _CHUNK) * _CHUNK
    nch = epw_c // _CHUNK
    d_half = d_feat // 2
    assert d_half % _LANES == 0
    assert n_nodes % _RB == 0
    rps = -(-(n_nodes // ns) // _RB) * _RB
    last_rows = n_nodes - rps * (ns - 1)
    assert 0 < last_rows <= rps and last_rows % _RB == 0

    mesh = plsc.VectorSubcoreMesh(core_axis_name="c", subcore_axis_name="s")

    @functools.partial(
        pl.kernel,
        mesh=mesh,
        out_type=jax.ShapeDtypeStruct((n_edges,), jnp.float32),
        scratch_types=[
            pltpu.VMEM_SHARED((n_nodes, d_half), jnp.int32),
            pltpu.VMEM((epw_c,), jnp.int32),
            pltpu.VMEM((epw_c,), jnp.int32),
            pltpu.VMEM((_CHUNK, d_half), jnp.int32),
            pltpu.VMEM((_CHUNK, d_half), jnp.int32),
            pltpu.VMEM((_CHUNK, d_half), jnp.int32),
            pltpu.VMEM((_CHUNK, d_half), jnp.int32),
            pltpu.VMEM((_CHUNK, d_half), jnp.int32),
            pltpu.VMEM((_CHUNK, d_half), jnp.int32),
            pltpu.VMEM((_RB, d_feat), jnp.float32),
            pltpu.VMEM((_RB, d_feat), jnp.float32),
            pltpu.VMEM((epw_c,), jnp.float32),
            pltpu.SemaphoreType.DMA,
            pltpu.SemaphoreType.DMA,
            pltpu.SemaphoreType.DMA,
            pltpu.SemaphoreType.DMA,
            pltpu.SemaphoreType.DMA,
            pltpu.SemaphoreType.DMA,
            pltpu.SemaphoreType.DMA,
        ],
    )
    def k(feat_hbm, src_hbm, dst_hbm, out_hbm,
          table, src_v, dst_v, u_0, v_0, u_1, v_1, u_2, v_2,
          st_a, st_b, out_v, sem_0, sem_1, sem_2,
          sem_la, sem_lb, sem_pa, sem_pb):
        sid = lax.axis_index("s")
        wid = sid * nc + lax.axis_index("c")
        base = wid * epw
        pltpu.sync_copy(src_hbm.at[pl.ds(base, epw)], src_v.at[pl.ds(0, epw)])
        pltpu.sync_copy(dst_hbm.at[pl.ds(base, epw)], dst_v.at[pl.ds(0, epw)])
        if epw_c > epw:
            zeros16 = jnp.zeros((_LANES,), jnp.int32)
            offs = list(range(epw, epw_c - _LANES + 1, 8))
            assert offs and offs[-1] + _LANES >= epw_c
            for off in offs:
                src_v[pl.ds(off, _LANES)] = zeros16
                dst_v[pl.ds(off, _LANES)] = zeros16
        lane = lax.iota(jnp.int32, _LANES)

        row0 = sid * rps
        nb = jnp.where(sid == ns - 1, last_rows // _RB, rps // _RB)
        half_bit = jnp.full((_LANES,), 0x8000, jnp.int32)
        hi_mask = jnp.full((_LANES,), -0x10000, jnp.int32)

        def ld_blk(blk, st, sem):
            return pltpu.make_async_copy(
                feat_hbm.at[pl.ds(row0 + blk * _RB, _RB)], st, sem)

        def wr_blk(blk, pk, sem):
            return pltpu.make_async_copy(
                pk.at[pl.ds(0, _RB)],
                table.at[pl.ds(row0 + blk * _RB, _RB)], sem)

        def pack_blk(st, pk):
            for r in range(_RB):
                for t in range(d_half // _LANES):
                    wl = st[r, pl.ds(t * _LANES, _LANES)]
                    wh = st[r, pl.ds(d_half + t * _LANES, _LANES)]
                    bl = lax.bitcast_convert_type(wl, jnp.int32)
                    bh = lax.bitcast_convert_type(wh, jnp.int32)
                    lo = lax.shift_right_logical(bl + half_bit, 16)
                    hi = (bh + half_bit) & hi_mask
                    pk[r, pl.ds(t * _LANES, _LANES)] = lo | hi

        ld_blk(0, st_a, sem_la).start()

        @pl.when(nb > 1)
        def _():
            ld_blk(1, st_b, sem_lb).start()

        def stage_pair(i, _):
            for p, (st, pk, sem_l, sem_p) in enumerate(
                    ((st_a, u_0, sem_la, sem_pa), (st_b, u_1, sem_lb, sem_pb))):
                blk = 2 * i + p

                @pl.when(blk < nb)
                def _():
                    ld_blk(blk, st, sem_l).wait()

                    @pl.when(blk >= 2)
                    def _():
                        wr_blk(blk - 2, pk, sem_p).wait()

                    pack_blk(st, pk)
                    wr_blk(blk, pk, sem_p).start()

                    @pl.when(blk + 2 < nb)
                    def _():
                        ld_blk(blk + 2, st, sem_l).start()

            return 0

        lax.fori_loop(0, (nb + 1) // 2, stage_pair, 0)

        @pl.when(nb >= 1)
        def _():
            wr_blk(0, u_0, sem_pa).wait()

        @pl.when(nb >= 2)
        def _():
            wr_blk(0, u_1, sem_pb).wait()

        plsc.subcore_barrier()

        def copies(ch, ub, vb, sem):
            off = pl.multiple_of(ch * _CHUNK, 8)
            cu = pltpu.make_async_copy(
                table.at[src_v.at[pl.ds(off, _CHUNK)]], ub, sem)
            cv = pltpu.make_async_copy(
                table.at[dst_v.at[pl.ds(off, _CHUNK)]], vb, sem)
            return cu, cv

        def fire(ch, ub, vb, sem):
            cu, cv = copies(ch, ub, vb, sem)
            cu.start()
            cv.start()

        def wait(ch, ub, vb, sem):
            cu, cv = copies(ch, ub, vb, sem)
            cu.wait()
            cv.wait()

        def compute(ch, ub, vb):
            off = pl.multiple_of(ch * _CHUNK, 8)

            def widen(wi):
                even = lax.bitcast_convert_type(wi << 16, jnp.float32)
                odd = lax.bitcast_convert_type(wi, jnp.float32)
                return even, odd

            def edge_partials(e):
                accs = [None] * 4
                for t in range(d_half // _LANES):
                    uw = ub[e, pl.ds(t * _LANES, _LANES)]
                    vw = vb[e, pl.ds(t * _LANES, _LANES)]
                    u0, u1 = widen(uw)
                    v0, v1 = widen(vw)
                    a, b = 2 * (t % 2), 2 * (t % 2) + 1
                    if accs[a] is None:
                        accs[a], accs[b] = u0 * v0, u1 * v1
                    else:
                        accs[a] = accs[a] + u0 * v0
                        accs[b] = accs[b] + u1 * v1
                return (accs[0] + accs[1]) + (accs[2] + accs[3])

            m8 = lane < 8
            for g in range(_CHUNK // _LANES):
                eb = g * _LANES

                def pair_body(j, res, eb=eb):
                    a = edge_partials(eb + j)
                    b = edge_partials(eb + j + 8)
                    c = (jnp.where(m8, a, b)
                         + _xlane(jnp.where(m8, b, a), lane ^ 8))
                    for hop in (4, 2, 1):
                        c = c + _xlane(c, lane ^ hop)
                    res = jnp.where(lane == j, c, res)
                    res = jnp.where(lane == j + 8, c, res)
                    return res

                res = lax.fori_loop(0, 8, pair_body,
                                    jnp.zeros((_LANES,), jnp.float32))
                out_v[pl.ds(off + eb, _LANES)] = res

        bufs = ((u_0, v_0, sem_0), (u_1, v_1, sem_1), (u_2, v_2, sem_2))
        for b, (ub, vb, sem) in enumerate(bufs):
            if b < nch:
                fire(b, ub, vb, sem)

        n_outer = -(-nch // _NBUF)

        def ring_chunks(i, _):
            for b, (ub, vb, sem) in enumerate(bufs):
                ch = _NBUF * i + b

                @pl.when(ch < nch)
                def _():
                    wait(ch, ub, vb, sem)
                    compute(ch, ub, vb)

                    @pl.when(ch + _NBUF < nch)
                    def _():
                        fire(ch + _NBUF, ub, vb, sem)

            return 0

        lax.fori_loop(0, n_outer, ring_chunks, 0)
        pltpu.sync_copy(out_v.at[pl.ds(0, epw)], out_hbm.at[pl.ds(base, epw)])

    return k


def kernel(feat, edge_index):
    n_nodes, d_feat = feat.shape
    n_edges = edge_index.shape[1]
    k = _make_sc_kernel(n_nodes, d_feat, n_edges)
    ei = edge_index.astype(jnp.int32)
    score = k(feat, ei[0], ei[1])
    return score.reshape(n_edges, 1)


# speedup vs baseline: 1.1355x; 1.1355x over previous
import functools

import jax
import jax.numpy as jnp
from jax import lax
from jax.experimental import pallas as pl
from jax.experimental.pallas import tpu as pltpu
from jax.experimental.pallas import tpu_sc as plsc

_LANES = 16
_CHUNK = 32
_NBUF = 3
_RB = 16

_DNUMS = lax.GatherDimensionNumbers(
    offset_dims=(), collapsed_slice_dims=(0,), start_index_map=(0,))


def _xlane(v, idx):
    return lax.gather(v, idx[:, None], dimension_numbers=_DNUMS,
                      slice_sizes=(1,),
                      mode=lax.GatherScatterMode.PROMISE_IN_BOUNDS)


@functools.lru_cache(maxsize=None)
def _make_sc_kernel(n_nodes, d_feat, n_edges):
    info = plsc.get_sparse_core_info()
    nc, ns = info.num_cores, info.num_subcores
    nw = nc * ns
    assert n_edges % (nw * 8) == 0
    epw = n_edges // nw
    epw_c = -(-epw // _CHUNK) * _CHUNK
    nch = epw_c // _CHUNK
    d_half = d_feat // 2
    assert d_half % _LANES == 0
    assert n_nodes % _RB == 0
    rps = -(-(n_nodes // ns) // _RB) * _RB
    last_rows = n_nodes - rps * (ns - 1)
    assert 0 < last_rows <= rps and last_rows % _RB == 0

    mesh = plsc.VectorSubcoreMesh(core_axis_name="c", subcore_axis_name="s")

    @functools.partial(
        pl.kernel,
        mesh=mesh,
        out_type=jax.ShapeDtypeStruct((n_edges,), jnp.float32),
        scratch_types=[
            pltpu.VMEM_SHARED((n_nodes, d_half), jnp.int32),
            pltpu.VMEM((epw_c,), jnp.int32),
            pltpu.VMEM((epw_c,), jnp.int32),
            pltpu.VMEM((_CHUNK, d_half), jnp.int32),
            pltpu.VMEM((_CHUNK, d_half), jnp.int32),
            pltpu.VMEM((_CHUNK, d_half), jnp.int32),
            pltpu.VMEM((_CHUNK, d_half), jnp.int32),
            pltpu.VMEM((_CHUNK, d_half), jnp.int32),
            pltpu.VMEM((_CHUNK, d_half), jnp.int32),
            pltpu.VMEM((_RB, d_feat), jnp.float32),
            pltpu.VMEM((_RB, d_feat), jnp.float32),
            pltpu.VMEM((epw_c,), jnp.float32),
            pltpu.SemaphoreType.DMA,
            pltpu.SemaphoreType.DMA,
            pltpu.SemaphoreType.DMA,
            pltpu.SemaphoreType.DMA,
            pltpu.SemaphoreType.DMA,
            pltpu.SemaphoreType.DMA,
            pltpu.SemaphoreType.DMA,
        ],
    )
    def k(feat_hbm, src_hbm, dst_hbm, out_hbm,
          table, src_v, dst_v, u_0, v_0, u_1, v_1, u_2, v_2,
          st_a, st_b, out_v, sem_0, sem_1, sem_2,
          sem_la, sem_lb, sem_pa, sem_pb):
        sid = lax.axis_index("s")
        wid = sid * nc + lax.axis_index("c")
        base = wid * epw
        pltpu.sync_copy(src_hbm.at[pl.ds(base, epw)], src_v.at[pl.ds(0, epw)])
        pltpu.sync_copy(dst_hbm.at[pl.ds(base, epw)], dst_v.at[pl.ds(0, epw)])
        if epw_c > epw:
            zeros16 = jnp.zeros((_LANES,), jnp.int32)
            offs = list(range(epw, epw_c - _LANES + 1, 8))
            assert offs and offs[-1] + _LANES >= epw_c
            for off in offs:
                src_v[pl.ds(off, _LANES)] = zeros16
                dst_v[pl.ds(off, _LANES)] = zeros16
        lane = lax.iota(jnp.int32, _LANES)

        row0 = sid * rps
        nb = jnp.where(sid == ns - 1, last_rows // _RB, rps // _RB)
        half_bit = jnp.full((_LANES,), 0x8000, jnp.int32)
        hi_mask = jnp.full((_LANES,), -0x10000, jnp.int32)

        def ld_blk(blk, st, sem):
            return pltpu.make_async_copy(
                feat_hbm.at[pl.ds(row0 + blk * _RB, _RB)], st, sem)

        def wr_blk(blk, pk, sem):
            return pltpu.make_async_copy(
                pk.at[pl.ds(0, _RB)],
                table.at[pl.ds(row0 + blk * _RB, _RB)], sem)

        def pack_blk(st, pk):
            for r in range(_RB):
                for t in range(d_half // _LANES):
                    wl = st[r, pl.ds(t * _LANES, _LANES)]
                    wh = st[r, pl.ds(d_half + t * _LANES, _LANES)]
                    bl = lax.bitcast_convert_type(wl, jnp.int32)
                    bh = lax.bitcast_convert_type(wh, jnp.int32)
                    lo = lax.shift_right_logical(bl + half_bit, 16)
                    hi = (bh + half_bit) & hi_mask
                    pk[r, pl.ds(t * _LANES, _LANES)] = lo | hi

        ld_blk(0, st_a, sem_la).start()

        @pl.when(nb > 1)
        def _():
            ld_blk(1, st_b, sem_lb).start()

        def stage_pair(i, _):
            for p, (st, pk, sem_l, sem_p) in enumerate(
                    ((st_a, u_0, sem_la, sem_pa), (st_b, u_1, sem_lb, sem_pb))):
                blk = 2 * i + p

                @pl.when(blk < nb)
                def _():
                    ld_blk(blk, st, sem_l).wait()

                    @pl.when(blk >= 2)
                    def _():
                        wr_blk(blk - 2, pk, sem_p).wait()

                    pack_blk(st, pk)
                    wr_blk(blk, pk, sem_p).start()

                    @pl.when(blk + 2 < nb)
                    def _():
                        ld_blk(blk + 2, st, sem_l).start()

            return 0

        lax.fori_loop(0, (nb + 1) // 2, stage_pair, 0)

        @pl.when(nb >= 1)
        def _():
            wr_blk(0, u_0, sem_pa).wait()

        @pl.when(nb >= 2)
        def _():
            wr_blk(0, u_1, sem_pb).wait()

        plsc.subcore_barrier()

        def copies(ch, ub, vb, sem):
            off = pl.multiple_of(ch * _CHUNK, 8)
            cu = pltpu.make_async_copy(
                table.at[src_v.at[pl.ds(off, _CHUNK)]], ub, sem)
            cv = pltpu.make_async_copy(
                table.at[dst_v.at[pl.ds(off, _CHUNK)]], vb, sem)
            return cu, cv

        def fire(ch, ub, vb, sem):
            cu, cv = copies(ch, ub, vb, sem)
            cu.start()
            cv.start()

        def wait(ch, ub, vb, sem):
            cu, cv = copies(ch, ub, vb, sem)
            cu.wait()
            cv.wait()

        def compute(ch, ub, vb):
            off = pl.multiple_of(ch * _CHUNK, 8)

            def widen(wi):
                even = lax.bitcast_convert_type(wi << 16, jnp.float32)
                odd = lax.bitcast_convert_type(wi, jnp.float32)
                return even, odd

            def edge_partials(e):
                acc0 = acc1 = None
                for t in range(d_half // _LANES):
                    uw = ub[e, pl.ds(t * _LANES, _LANES)]
                    vw = vb[e, pl.ds(t * _LANES, _LANES)]
                    u0, u1 = widen(uw)
                    v0, v1 = widen(vw)
                    if acc0 is None:
                        acc0, acc1 = u0 * v0, u1 * v1
                    else:
                        acc0 = acc0 + u0 * v0
                        acc1 = acc1 + u1 * v1
                return acc0 + acc1

            m8 = lane < 8
            for g in range(_CHUNK // _LANES):
                eb = g * _LANES

                def pair_body(j, res, eb=eb):
                    a = edge_partials(eb + j)
                    b = edge_partials(eb + j + 8)
                    c = (jnp.where(m8, a, b)
                         + _xlane(jnp.where(m8, b, a), lane ^ 8))
                    for hop in (4, 2, 1):
                        c = c + _xlane(c, lane ^ hop)
                    res = jnp.where(lane == j, c, res)
                    res = jnp.where(lane == j + 8, c, res)
                    return res

                res = lax.fori_loop(0, 8, pair_body,
                                    jnp.zeros((_LANES,), jnp.float32))
                out_v[pl.ds(off + eb, _LANES)] = res

        bufs = ((u_0, v_0, sem_0), (u_1, v_1, sem_1), (u_2, v_2, sem_2))
        for b, (ub, vb, sem) in enumerate(bufs):
            if b < nch:
                fire(b, ub, vb, sem)

        n_outer = -(-nch // _NBUF)

        def ring_chunks(i, _):
            for b, (ub, vb, sem) in enumerate(bufs):
                ch = _NBUF * i + b

                @pl.when(ch < nch)
                def _():
                    wait(ch, ub, vb, sem)
                    compute(ch, ub, vb)

                    @pl.when(ch + _NBUF < nch)
                    def _():
                        fire(ch + _NBUF, ub, vb, sem)

            return 0

        lax.fori_loop(0, n_outer, ring_chunks, 0)
        pltpu.sync_copy(out_v.at[pl.ds(0, epw)], out_hbm.at[pl.ds(base, epw)])

    return k


def kernel(feat, edge_index):
    n_nodes, d_feat = feat.shape
    n_edges = edge_index.shape[1]
    k = _make_sc_kernel(n_nodes, d_feat, n_edges)
    ei = edge_index.astype(jnp.int32)
    score = k(feat, ei[0], ei[1])
    return score.reshape(n_edges, 1)


# speedup vs baseline: 1.1399x; 1.0039x over previous
import functools

import jax
import jax.numpy as jnp
from jax import lax
from jax.experimental import pallas as pl
from jax.experimental.pallas import tpu as pltpu
from jax.experimental.pallas import tpu_sc as plsc

_LANES = 16
_CHUNK = 32
_NBUF = 3
_RB = 16

_DNUMS = lax.GatherDimensionNumbers(
    offset_dims=(), collapsed_slice_dims=(0,), start_index_map=(0,))


def _xlane(v, idx):
    return lax.gather(v, idx[:, None], dimension_numbers=_DNUMS,
                      slice_sizes=(1,),
                      mode=lax.GatherScatterMode.PROMISE_IN_BOUNDS)


@functools.lru_cache(maxsize=None)
def _make_sc_kernel(n_nodes, d_feat, n_edges):
    info = plsc.get_sparse_core_info()
    nc, ns = info.num_cores, info.num_subcores
    nw = nc * ns
    assert n_edges % (nw * 8) == 0
    epw = n_edges // nw
    epw_c = -(-epw // _CHUNK) * _CHUNK
    nch = epw_c // _CHUNK
    d_half = d_feat // 2
    assert d_half % _LANES == 0
    assert n_nodes % _RB == 0
    rps = -(-(n_nodes // ns) // _RB) * _RB
    last_rows = n_nodes - rps * (ns - 1)
    assert 0 < last_rows <= rps and last_rows % _RB == 0

    mesh = plsc.VectorSubcoreMesh(core_axis_name="c", subcore_axis_name="s")

    @functools.partial(
        pl.kernel,
        mesh=mesh,
        out_type=jax.ShapeDtypeStruct((n_edges,), jnp.float32),
        scratch_types=[
            pltpu.VMEM_SHARED((n_nodes, d_half), jnp.int32),
            pltpu.VMEM((epw_c,), jnp.int32),
            pltpu.VMEM((epw_c,), jnp.int32),
            pltpu.VMEM((_CHUNK, d_half), jnp.int32),
            pltpu.VMEM((_CHUNK, d_half), jnp.int32),
            pltpu.VMEM((_CHUNK, d_half), jnp.int32),
            pltpu.VMEM((_CHUNK, d_half), jnp.int32),
            pltpu.VMEM((_CHUNK, d_half), jnp.int32),
            pltpu.VMEM((_CHUNK, d_half), jnp.int32),
            pltpu.VMEM((_RB, d_feat), jnp.float32),
            pltpu.VMEM((_RB, d_feat), jnp.float32),
            pltpu.VMEM((epw_c,), jnp.float32),
            pltpu.SemaphoreType.DMA,
            pltpu.SemaphoreType.DMA,
            pltpu.SemaphoreType.DMA,
            pltpu.SemaphoreType.DMA,
            pltpu.SemaphoreType.DMA,
            pltpu.SemaphoreType.DMA,
            pltpu.SemaphoreType.DMA,
        ],
    )
    def k(feat_hbm, src_hbm, dst_hbm, out_hbm,
          table, src_v, dst_v, u_0, v_0, u_1, v_1, u_2, v_2,
          st_a, st_b, out_v, sem_0, sem_1, sem_2,
          sem_la, sem_lb, sem_pa, sem_pb):
        sid = lax.axis_index("s")
        wid = sid * nc + lax.axis_index("c")
        base = wid * epw
        lane = lax.iota(jnp.int32, _LANES)

        row0 = sid * rps
        nb = jnp.where(sid == ns - 1, last_rows // _RB, rps // _RB)
        half_bit = jnp.full((_LANES,), 0x8000, jnp.int32)
        hi_mask = jnp.full((_LANES,), -0x10000, jnp.int32)

        def ld_blk(blk, st, sem):
            return pltpu.make_async_copy(
                feat_hbm.at[pl.ds(row0 + blk * _RB, _RB)], st, sem)

        def wr_blk(blk, pk, sem):
            return pltpu.make_async_copy(
                pk.at[pl.ds(0, _RB)],
                table.at[pl.ds(row0 + blk * _RB, _RB)], sem)

        def pack_blk(st, pk):
            for r in range(_RB):
                for t in range(d_half // _LANES):
                    wl = st[r, pl.ds(t * _LANES, _LANES)]
                    wh = st[r, pl.ds(d_half + t * _LANES, _LANES)]
                    bl = lax.bitcast_convert_type(wl, jnp.int32)
                    bh = lax.bitcast_convert_type(wh, jnp.int32)
                    lo = lax.shift_right_logical(bl + half_bit, 16)
                    hi = (bh + half_bit) & hi_mask
                    pk[r, pl.ds(t * _LANES, _LANES)] = lo | hi

        ld_blk(0, st_a, sem_la).start()

        @pl.when(nb > 1)
        def _():
            ld_blk(1, st_b, sem_lb).start()

        pltpu.sync_copy(src_hbm.at[pl.ds(base, epw)], src_v.at[pl.ds(0, epw)])
        pltpu.sync_copy(dst_hbm.at[pl.ds(base, epw)], dst_v.at[pl.ds(0, epw)])
        if epw_c > epw:
            zeros16 = jnp.zeros((_LANES,), jnp.int32)
            offs = list(range(epw, epw_c - _LANES + 1, 8))
            assert offs and offs[-1] + _LANES >= epw_c
            for off in offs:
                src_v[pl.ds(off, _LANES)] = zeros16
                dst_v[pl.ds(off, _LANES)] = zeros16

        def stage_pair(i, _):
            for p, (st, pk, sem_l, sem_p) in enumerate(
                    ((st_a, u_0, sem_la, sem_pa), (st_b, u_1, sem_lb, sem_pb))):
                blk = 2 * i + p

                @pl.when(blk < nb)
                def _():
                    ld_blk(blk, st, sem_l).wait()

                    @pl.when(blk >= 2)
                    def _():
                        wr_blk(blk - 2, pk, sem_p).wait()

                    pack_blk(st, pk)
                    wr_blk(blk, pk, sem_p).start()

                    @pl.when(blk + 2 < nb)
                    def _():
                        ld_blk(blk + 2, st, sem_l).start()

            return 0

        lax.fori_loop(0, (nb + 1) // 2, stage_pair, 0)

        @pl.when(nb >= 1)
        def _():
            wr_blk(0, u_0, sem_pa).wait()

        @pl.when(nb >= 2)
        def _():
            wr_blk(0, u_1, sem_pb).wait()

        plsc.subcore_barrier()

        def copies(ch, ub, vb, sem):
            off = pl.multiple_of(ch * _CHUNK, 8)
            cu = pltpu.make_async_copy(
                table.at[src_v.at[pl.ds(off, _CHUNK)]], ub, sem)
            cv = pltpu.make_async_copy(
                table.at[dst_v.at[pl.ds(off, _CHUNK)]], vb, sem)
            return cu, cv

        def fire(ch, ub, vb, sem):
            cu, cv = copies(ch, ub, vb, sem)
            cu.start()
            cv.start()

        def wait(ch, ub, vb, sem):
            cu, cv = copies(ch, ub, vb, sem)
            cu.wait()
            cv.wait()

        def compute(ch, ub, vb):
            off = pl.multiple_of(ch * _CHUNK, 8)

            def widen(wi):
                even = lax.bitcast_convert_type(wi << 16, jnp.float32)
                odd = lax.bitcast_convert_type(wi, jnp.float32)
                return even, odd

            def edge_partials(e):
                acc0 = acc1 = None
                for t in range(d_half // _LANES):
                    uw = ub[e, pl.ds(t * _LANES, _LANES)]
                    vw = vb[e, pl.ds(t * _LANES, _LANES)]
                    u0, u1 = widen(uw)
                    v0, v1 = widen(vw)
                    if acc0 is None:
                        acc0, acc1 = u0 * v0, u1 * v1
                    else:
                        acc0 = acc0 + u0 * v0
                        acc1 = acc1 + u1 * v1
                return acc0 + acc1

            m8 = lane < 8
            for g in range(_CHUNK // _LANES):
                eb = g * _LANES

                def pair_body(j, res, eb=eb):
                    a = edge_partials(eb + j)
                    b = edge_partials(eb + j + 8)
                    c = (jnp.where(m8, a, b)
                         + _xlane(jnp.where(m8, b, a), lane ^ 8))
                    for hop in (4, 2, 1):
                        c = c + _xlane(c, lane ^ hop)
                    res = jnp.where(lane == j, c, res)
                    res = jnp.where(lane == j + 8, c, res)
                    return res

                res = lax.fori_loop(0, 8, pair_body,
                                    jnp.zeros((_LANES,), jnp.float32))
                out_v[pl.ds(off + eb, _LANES)] = res

        bufs = ((u_0, v_0, sem_0), (u_1, v_1, sem_1), (u_2, v_2, sem_2))
        for b, (ub, vb, sem) in enumerate(bufs):
            if b < nch:
                fire(b, ub, vb, sem)

        n_outer = -(-nch // _NBUF)

        def ring_chunks(i, _):
            for b, (ub, vb, sem) in enumerate(bufs):
                ch = _NBUF * i + b

                @pl.when(ch < nch)
                def _():
                    wait(ch, ub, vb, sem)
                    compute(ch, ub, vb)

                    @pl.when(ch + _NBUF < nch)
                    def _():
                        fire(ch + _NBUF, ub, vb, sem)

            return 0

        lax.fori_loop(0, n_outer, ring_chunks, 0)
        pltpu.sync_copy(out_v.at[pl.ds(0, epw)], out_hbm.at[pl.ds(base, epw)])

    return k


def kernel(feat, edge_index):
    n_nodes, d_feat = feat.shape
    n_edges = edge_index.shape[1]
    k = _make_sc_kernel(n_nodes, d_feat, n_edges)
    ei = edge_index.astype(jnp.int32)
    score = k(feat, ei[0], ei[1])
    return score.reshape(n_edges, 1)
